# baseline (speedup 1.0000x reference)
import functools
import jax
import jax.numpy as jnp
from jax.experimental import pallas as pl
from jax.experimental.pallas import tpu as pltpu

_BF = jnp.bfloat16
_F32 = jnp.float32


def _dec2(v):
    b, h2, w2, c = v.shape
    v6 = v.reshape(b, h2 // 2, 2, w2 // 2, 2, c)
    return v6[:, :, 0, :, 0, :]


def _bneck(x, refs, s3_ref, H, W, stride, P, has_down):
    w1_ref, b1_ref, w2_ref, b2_ref, w3_ref, b3_ref = refs[:6]
    Bb, _, _, Cin = x.shape
    oh, ow = H // stride, W // stride
    xm = x.reshape(Bb * H * W, Cin)
    y1 = jnp.dot(xm, w1_ref[...], preferred_element_type=_F32) + b1_ref[...]
    y1 = jnp.maximum(y1, 0.0).astype(_BF).reshape(Bb, H, W, P)
    s3_ref[:, 0:H + 2, 0:W, :] = jnp.zeros((Bb, H + 2, W, 3 * P), _BF)
    s3_ref[:, 1:H + 1, 1:W, 0:P] = y1[:, :, 0:W - 1, :]
    s3_ref[:, 1:H + 1, 0:W, P:2 * P] = y1
    s3_ref[:, 1:H + 1, 0:W - 1, 2 * P:3 * P] = y1[:, :, 1:W, :]
    acc = b2_ref[...]
    for di in range(3):
        tap = s3_ref[:, di:di + H, 0:W, :].reshape(Bb * H * W, 3 * P)
        acc = acc + jnp.dot(tap, w2_ref[di * 3 * P:(di + 1) * 3 * P, :],
                            preferred_element_type=_F32)
    y2 = jnp.maximum(acc, 0.0).astype(_BF)
    if stride == 2:
        y2 = _dec2(y2.reshape(Bb, H, W, P)).reshape(Bb * oh * ow, P)
    z = jnp.dot(y2, w3_ref[...], preferred_element_type=_F32) + b3_ref[...]
    if has_down:
        wd_ref, bd_ref = refs[6:8]
        xs = _dec2(x) if stride == 2 else x
        scm = xs.reshape(Bb * oh * ow, Cin)
        z = z + jnp.dot(scm, wd_ref[...], preferred_element_type=_F32) + bd_ref[...]
    else:
        z = z + xm.astype(_F32)
    Cout = w3_ref.shape[1]
    return jnp.maximum(z, 0.0).astype(_BF).reshape(Bb, oh, ow, Cout)


def _layer_kernel(x_ref, *rest, H, W, s0, P, nb):
    refs = rest[:8 + 6 * (nb - 1)]
    o_ref, s3_ref = rest[8 + 6 * (nb - 1):]
    x = x_ref[...]
    x = _bneck(x, refs[0:8], s3_ref, H, W, s0, P, True)
    oh, ow = H // s0, W // s0
    for k in range(nb - 1):
        x = _bneck(x, refs[8 + 6 * k:8 + 6 * (k + 1)], s3_ref,
                   oh, ow, 1, P, False)
    o_ref[...] = x


def _run_layer(x, blocks, s0, Bb):
    B, H, W, Cin = x.shape
    P = blocks[0][0].shape[1]
    Cout = blocks[0][4].shape[1]
    oh, ow = H // s0, W // s0
    nb = len(blocks)
    args = [x] + [a for blk in blocks for a in blk]
    in_specs = [pl.BlockSpec((Bb, H, W, Cin), lambda i: (i, 0, 0, 0))]
    for blk in blocks:
        in_specs += [pl.BlockSpec(a.shape, lambda i: (0, 0)) for a in blk]
    kern = functools.partial(_layer_kernel, H=H, W=W, s0=s0, P=P, nb=nb)
    return pl.pallas_call(
        kern,
        out_shape=jax.ShapeDtypeStruct((B, oh, ow, Cout), _BF),
        grid_spec=pltpu.PrefetchScalarGridSpec(
            num_scalar_prefetch=0, grid=(B // Bb,),
            in_specs=in_specs,
            out_specs=pl.BlockSpec((Bb, oh, ow, Cout), lambda i: (i, 0, 0, 0)),
            scratch_shapes=[pltpu.VMEM((Bb, H + 2, W, 3 * P), _BF)]),
        compiler_params=pltpu.CompilerParams(
            dimension_semantics=("parallel",),
            vmem_limit_bytes=64 * 1024 * 1024),
    )(*args)


def _stem_kernel(z_ref, w_ref, b_ref, o_ref, s_ref, ip_ref, jp_ref):
    zz = z_ref[...]
    Bb = zz.shape[0]
    s_ref[...] = jnp.zeros((Bb, 67, 64, 64), _BF)
    s_ref[:, 2:66, 2:64, 0:12] = zz[:, :, 0:62, :]
    s_ref[:, 2:66, 1:64, 16:28] = zz[:, :, 0:63, :]
    s_ref[:, 2:66, 0:64, 32:44] = zz
    s_ref[:, 2:66, 0:63, 48:60] = zz[:, :, 1:64, :]
    acc = b_ref[...]
    for q in range(4):
        tap = s_ref[:, q:q + 64, :, :].reshape(Bb * 64 * 64, 64)
        acc = acc + jnp.dot(tap, w_ref[q * 64:(q + 1) * 64, :],
                            preferred_element_type=_F32)
    y = jnp.maximum(acc, 0.0).astype(_BF).reshape(Bb, 64, 64, 128)
    ip_ref[...] = jnp.full((Bb, 66, 64, 128), -jnp.inf, _BF)
    ip_ref[:, 1:65, :, :] = y
    a = jnp.maximum(jnp.maximum(ip_ref[:, 0:64, :, :], ip_ref[:, 1:65, :, :]),
                    ip_ref[:, 2:66, :, :])
    d = a.reshape(Bb, 32, 2, 64, 128)[:, :, 0]
    jp_ref[...] = jnp.full((Bb, 32, 66, 128), -jnp.inf, _BF)
    jp_ref[:, :, 1:65, :] = d
    e = jp_ref[...].reshape(Bb, 32, 33, 2, 128)
    out = jnp.maximum(jnp.maximum(e[:, :, 0:32, 0], e[:, :, 0:32, 1]),
                      e[:, :, 1:33, 0])
    o_ref[...] = out.reshape(Bb, 32, 32, 128)


def _run_stem(z, wz, b, Bb=4):
    B = z.shape[0]
    return pl.pallas_call(
        _stem_kernel,
        out_shape=jax.ShapeDtypeStruct((B, 32, 32, 128), _BF),
        grid_spec=pltpu.PrefetchScalarGridSpec(
            num_scalar_prefetch=0, grid=(B // Bb,),
            in_specs=[pl.BlockSpec((Bb, 64, 64, 12), lambda i: (i, 0, 0, 0)),
                      pl.BlockSpec(wz.shape, lambda i: (0, 0)),
                      pl.BlockSpec(b.shape, lambda i: (0, 0))],
            out_specs=pl.BlockSpec((Bb, 32, 32, 128), lambda i: (i, 0, 0, 0)),
            scratch_shapes=[pltpu.VMEM((Bb, 67, 64, 64), _BF),
                            pltpu.VMEM((Bb, 66, 64, 128), _BF),
                            pltpu.VMEM((Bb, 32, 66, 128), _BF)]),
        compiler_params=pltpu.CompilerParams(
            dimension_semantics=("parallel",)),
    )(z, wz, b)


def _head_kernel(x_ref, w1a, b1a, w1b, b1b, w2a, b2a, w2b, b2b, wl, bl,
                 o_ref, *, pool):
    if pool:
        xm = x_ref[...].reshape(8 * 16, 2048).astype(_F32)
        h = jnp.mean(xm.reshape(8, 16, 2048), axis=1)
    else:
        s = x_ref.shape
        h = x_ref[...].reshape(s[0] * s[1] * s[2], s[3]).astype(_F32)

    def bconv(hh, wa, ba, wb, bb):
        y = jnp.dot(hh.astype(_BF), wa[...], preferred_element_type=_F32) + ba[...]
        y = jnp.maximum(y, 0.0)
        zz = jnp.dot(y.astype(_BF), wb[...], preferred_element_type=_F32) + bb[...]
        return jnp.maximum(zz + hh, 0.0)

    h = bconv(h, w1a, b1a, w1b, b1b)
    h = bconv(h, w2a, b2a, w2b, b2b)
    o_ref[...] = jnp.dot(h.astype(_BF), wl[...], preferred_element_type=_F32) + bl[...]


def _run_global_head(x, ws):
    return pl.pallas_call(
        functools.partial(_head_kernel, pool=True),
        out_shape=jax.ShapeDtypeStruct((8, 128), _F32),
        compiler_params=pltpu.CompilerParams(
            vmem_limit_bytes=48 * 1024 * 1024),
    )(x, *ws)


def _run_local_head(x, ws):
    B, H, W, C = x.shape
    Bb = 4
    M = Bb * H * W
    in_specs = [pl.BlockSpec((Bb, H, W, C), lambda i: (i, 0, 0, 0))]
    in_specs += [pl.BlockSpec(w.shape, lambda i: (0, 0)) for w in ws]
    return pl.pallas_call(
        functools.partial(_head_kernel, pool=False),
        out_shape=jax.ShapeDtypeStruct((B * H * W, 128), _F32),
        grid_spec=pltpu.PrefetchScalarGridSpec(
            num_scalar_prefetch=0, grid=(B // Bb,),
            in_specs=in_specs,
            out_specs=pl.BlockSpec((M, 128), lambda i: (i, 0))),
        compiler_params=pltpu.CompilerParams(
            dimension_semantics=("parallel",)),
    )(x, *ws)


_LAYER_CFG = ((0, 3, 1), (1, 4, 2), (2, 6, 2), (3, 3, 2))
_BB_FOR_LAYER = {0: 2, 1: 4, 2: 8, 3: 8}


def kernel(conv1_w, conv1_b, l0b0_conv1_w, l0b0_conv1_b, l0b0_conv2_w, l0b0_conv2_b, l0b0_conv3_w, l0b0_conv3_b, l0b0_down_w, l0b0_down_b, l0b1_conv1_w, l0b1_conv1_b, l0b1_conv2_w, l0b1_conv2_b, l0b1_conv3_w, l0b1_conv3_b, l0b2_conv1_w, l0b2_conv1_b, l0b2_conv2_w, l0b2_conv2_b, l0b2_conv3_w, l0b2_conv3_b, l1b0_conv1_w, l1b0_conv1_b, l1b0_conv2_w, l1b0_conv2_b, l1b0_conv3_w, l1b0_conv3_b, l1b0_down_w, l1b0_down_b, l1b1_conv1_w, l1b1_conv1_b, l1b1_conv2_w, l1b1_conv2_b, l1b1_conv3_w, l1b1_conv3_b, l1b2_conv1_w, l1b2_conv1_b, l1b2_conv2_w, l1b2_conv2_b, l1b2_conv3_w, l1b2_conv3_b, l1b3_conv1_w, l1b3_conv1_b, l1b3_conv2_w, l1b3_conv2_b, l1b3_conv3_w, l1b3_conv3_b, l2b0_conv1_w, l2b0_conv1_b, l2b0_conv2_w, l2b0_conv2_b, l2b0_conv3_w, l2b0_conv3_b, l2b0_down_w, l2b0_down_b, l2b1_conv1_w, l2b1_conv1_b, l2b1_conv2_w, l2b1_conv2_b, l2b1_conv3_w, l2b1_conv3_b, l2b2_conv1_w, l2b2_conv1_b, l2b2_conv2_w, l2b2_conv2_b, l2b2_conv3_w, l2b2_conv3_b, l2b3_conv1_w, l2b3_conv1_b, l2b3_conv2_w, l2b3_conv2_b, l2b3_conv3_w, l2b3_conv3_b, l2b4_conv1_w, l2b4_conv1_b, l2b4_conv2_w, l2b4_conv2_b, l2b4_conv3_w, l2b4_conv3_b, l2b5_conv1_w, l2b5_conv1_b, l2b5_conv2_w, l2b5_conv2_b, l2b5_conv3_w, l2b5_conv3_b, l3b0_conv1_w, l3b0_conv1_b, l3b0_conv2_w, l3b0_conv2_b, l3b0_conv3_w, l3b0_conv3_b, l3b0_down_w, l3b0_down_b, l3b1_conv1_w, l3b1_conv1_b, l3b1_conv2_w, l3b1_conv2_b, l3b1_conv3_w, l3b1_conv3_b, l3b2_conv1_w, l3b2_conv1_b, l3b2_conv2_w, l3b2_conv2_b, l3b2_conv3_w, l3b2_conv3_b, fc_bn1_c1_w, fc_bn1_c1_b, fc_bn1_c2_w, fc_bn1_c2_b, fc_bn2_c1_w, fc_bn2_c1_b, fc_bn2_c2_w, fc_bn2_c2_b, fc_lin_w, fc_lin_b, proj_bn1_c1_w, proj_bn1_c1_b, proj_bn1_c2_w, proj_bn1_c2_b, proj_bn2_c1_w, proj_bn2_c1_b, proj_bn2_c2_w, proj_bn2_c2_b, proj_conv_w, proj_conv_b, coord_obj, mask_obj):
    flat = dict(locals())

    x = (coord_obj * mask_obj).astype(_BF)
    z = x.reshape(8, 3, 64, 2, 64, 2).transpose(0, 2, 4, 3, 5, 1)
    z = z.reshape(8, 64, 64, 12)
    w7 = conv1_w[:147].reshape(7, 7, 3, 128)
    w7p = jnp.pad(w7, ((1, 0), (1, 0), (0, 0), (0, 0)))
    wz = w7p.reshape(4, 2, 4, 2, 3, 128).transpose(0, 2, 1, 3, 4, 5)
    wz = jnp.pad(wz.reshape(4, 4, 12, 128), ((0, 0), (0, 0), (0, 4), (0, 0)))
    wz = wz.reshape(256, 128)
    x = _run_stem(z, wz, conv1_b)

    seen = None
    for li, nb, s0 in _LAYER_CFG:
        blocks = []
        for bi in range(nb):
            pre = "l%db%d_" % (li, bi)
            blk = (flat[pre + "conv1_w"], flat[pre + "conv1_b"],
                   flat[pre + "conv2_w"], flat[pre + "conv2_b"],
                   flat[pre + "conv3_w"], flat[pre + "conv3_b"])
            if bi == 0:
                blk = blk + (flat[pre + "down_w"], flat[pre + "down_b"])
            blocks.append(blk)
        x = _run_layer(x, blocks, s0, _BB_FOR_LAYER[li])
        if li == 2:
            seen = x

    g = _run_global_head(
        x, (fc_bn1_c1_w, fc_bn1_c1_b, fc_bn1_c2_w, fc_bn1_c2_b,
            fc_bn2_c1_w, fc_bn2_c1_b, fc_bn2_c2_w, fc_bn2_c2_b,
            fc_lin_w, fc_lin_b))
    f = _run_local_head(
        seen, (proj_bn1_c1_w, proj_bn1_c1_b, proj_bn1_c2_w, proj_bn1_c2_b,
               proj_bn2_c1_w, proj_bn2_c1_b, proj_bn2_c2_w, proj_bn2_c2_b,
               proj_conv_w, proj_conv_b))
    return jnp.concatenate([g[:, None, :], f.reshape(8, 64, 128)], axis=1)


# speedup vs baseline: 1.0629x; 1.0629x over previous
import functools
import jax
import jax.numpy as jnp
from jax.experimental import pallas as pl
from jax.experimental.pallas import tpu as pltpu

_BF = jnp.bfloat16
_F32 = jnp.float32


def _dec2(v):
    b, h2, w2, c = v.shape
    v6 = v.reshape(b, h2 // 2, 2, w2 // 2, 2, c)
    return v6[:, :, 0, :, 0, :]


def _bneck(x, refs, s3_ref, H, W, stride, P, has_down):
    w1_ref, b1_ref, w2_ref, b2_ref, w3_ref, b3_ref = refs[:6]
    Bb, _, _, Cin = x.shape
    oh, ow = H // stride, W // stride
    xm = x.reshape(Bb * H * W, Cin)
    y1 = jnp.dot(xm, w1_ref[...], preferred_element_type=_F32) + b1_ref[...]
    y1 = jnp.maximum(y1, 0.0).astype(_BF).reshape(Bb, H, W, P)
    s3_ref[:, 0:H + 2, 0:W, :] = jnp.zeros((Bb, H + 2, W, 3 * P), _BF)
    s3_ref[:, 1:H + 1, 1:W, 0:P] = y1[:, :, 0:W - 1, :]
    s3_ref[:, 1:H + 1, 0:W, P:2 * P] = y1
    s3_ref[:, 1:H + 1, 0:W - 1, 2 * P:3 * P] = y1[:, :, 1:W, :]
    acc = b2_ref[...]
    for di in range(3):
        tap = s3_ref[:, di:di + H, 0:W, :].reshape(Bb * H * W, 3 * P)
        acc = acc + jnp.dot(tap, w2_ref[di * 3 * P:(di + 1) * 3 * P, :],
                            preferred_element_type=_F32)
    y2 = jnp.maximum(acc, 0.0).astype(_BF)
    if stride == 2:
        y2 = _dec2(y2.reshape(Bb, H, W, P)).reshape(Bb * oh * ow, P)
    z = jnp.dot(y2, w3_ref[...], preferred_element_type=_F32) + b3_ref[...]
    if has_down:
        wd_ref, bd_ref = refs[6:8]
        xs = _dec2(x) if stride == 2 else x
        scm = xs.reshape(Bb * oh * ow, Cin)
        z = z + jnp.dot(scm, wd_ref[...], preferred_element_type=_F32) + bd_ref[...]
    else:
        z = z + xm.astype(_F32)
    Cout = w3_ref.shape[1]
    return jnp.maximum(z, 0.0).astype(_BF).reshape(Bb, oh, ow, Cout)


def _layer_kernel(x_ref, *rest, H, W, s0, P, nb):
    refs = rest[:8 + 6 * (nb - 1)]
    o_ref, s3_ref = rest[8 + 6 * (nb - 1):]
    x = x_ref[...]
    x = _bneck(x, refs[0:8], s3_ref, H, W, s0, P, True)
    oh, ow = H // s0, W // s0
    for k in range(nb - 1):
        x = _bneck(x, refs[8 + 6 * k:8 + 6 * (k + 1)], s3_ref,
                   oh, ow, 1, P, False)
    o_ref[...] = x


def _run_layer(x, blocks, s0, Bb):
    B, H, W, Cin = x.shape
    P = blocks[0][0].shape[1]
    Cout = blocks[0][4].shape[1]
    oh, ow = H // s0, W // s0
    nb = len(blocks)
    args = [x] + [a for blk in blocks for a in blk]
    in_specs = [pl.BlockSpec((Bb, H, W, Cin), lambda i: (i, 0, 0, 0))]
    for blk in blocks:
        in_specs += [pl.BlockSpec(a.shape, lambda i: (0, 0)) for a in blk]
    kern = functools.partial(_layer_kernel, H=H, W=W, s0=s0, P=P, nb=nb)
    return pl.pallas_call(
        kern,
        out_shape=jax.ShapeDtypeStruct((B, oh, ow, Cout), _BF),
        grid_spec=pltpu.PrefetchScalarGridSpec(
            num_scalar_prefetch=0, grid=(B // Bb,),
            in_specs=in_specs,
            out_specs=pl.BlockSpec((Bb, oh, ow, Cout), lambda i: (i, 0, 0, 0)),
            scratch_shapes=[pltpu.VMEM((Bb, H + 2, W, 3 * P), _BF)]),
        compiler_params=pltpu.CompilerParams(
            dimension_semantics=("parallel",),
            vmem_limit_bytes=64 * 1024 * 1024),
    )(*args)


def _stem_kernel(z_ref, w_ref, b_ref, o_ref, s_ref, ip_ref, jp_ref):
    zz = z_ref[...]
    Bb = zz.shape[0]
    s_ref[...] = jnp.zeros((Bb, 67, 64, 64), _BF)
    s_ref[:, 2:66, 2:64, 0:12] = zz[:, :, 0:62, :]
    s_ref[:, 2:66, 1:64, 16:28] = zz[:, :, 0:63, :]
    s_ref[:, 2:66, 0:64, 32:44] = zz
    s_ref[:, 2:66, 0:63, 48:60] = zz[:, :, 1:64, :]
    acc = b_ref[...]
    for q in range(4):
        tap = s_ref[:, q:q + 64, :, :].reshape(Bb * 64 * 64, 64)
        acc = acc + jnp.dot(tap, w_ref[q * 64:(q + 1) * 64, :],
                            preferred_element_type=_F32)
    y = jnp.maximum(acc, 0.0).astype(_BF).reshape(Bb, 64, 64, 128)
    ip_ref[...] = jnp.full((Bb, 66, 64, 128), -jnp.inf, _BF)
    ip_ref[:, 1:65, :, :] = y
    a = jnp.maximum(jnp.maximum(ip_ref[:, 0:64, :, :], ip_ref[:, 1:65, :, :]),
                    ip_ref[:, 2:66, :, :])
    d = a.reshape(Bb, 32, 2, 64, 128)[:, :, 0]
    jp_ref[...] = jnp.full((Bb, 32, 66, 128), -jnp.inf, _BF)
    jp_ref[:, :, 1:65, :] = d
    e = jp_ref[...].reshape(Bb, 32, 33, 2, 128)
    out = jnp.maximum(jnp.maximum(e[:, :, 0:32, 0], e[:, :, 0:32, 1]),
                      e[:, :, 1:33, 0])
    o_ref[...] = out.reshape(Bb, 32, 32, 128)


def _run_stem(z, wz, b, Bb=1):
    B = z.shape[0]
    return pl.pallas_call(
        _stem_kernel,
        out_shape=jax.ShapeDtypeStruct((B, 32, 32, 128), _BF),
        grid_spec=pltpu.PrefetchScalarGridSpec(
            num_scalar_prefetch=0, grid=(B // Bb,),
            in_specs=[pl.BlockSpec((Bb, 64, 64, 12), lambda i: (i, 0, 0, 0)),
                      pl.BlockSpec(wz.shape, lambda i: (0, 0)),
                      pl.BlockSpec(b.shape, lambda i: (0, 0))],
            out_specs=pl.BlockSpec((Bb, 32, 32, 128), lambda i: (i, 0, 0, 0)),
            scratch_shapes=[pltpu.VMEM((Bb, 67, 64, 64), _BF),
                            pltpu.VMEM((Bb, 66, 64, 128), _BF),
                            pltpu.VMEM((Bb, 32, 66, 128), _BF)]),
        compiler_params=pltpu.CompilerParams(
            dimension_semantics=("parallel",)),
    )(z, wz, b)


def _head_kernel(x_ref, w1a, b1a, w1b, b1b, w2a, b2a, w2b, b2b, wl, bl,
                 o_ref, *, pool):
    if pool:
        xm = x_ref[...].reshape(8 * 16, 2048).astype(_F32)
        h = jnp.mean(xm.reshape(8, 16, 2048), axis=1)
    else:
        s = x_ref.shape
        h = x_ref[...].reshape(s[0] * s[1] * s[2], s[3]).astype(_F32)

    def bconv(hh, wa, ba, wb, bb):
        y = jnp.dot(hh.astype(_BF), wa[...], preferred_element_type=_F32) + ba[...]
        y = jnp.maximum(y, 0.0)
        zz = jnp.dot(y.astype(_BF), wb[...], preferred_element_type=_F32) + bb[...]
        return jnp.maximum(zz + hh, 0.0)

    h = bconv(h, w1a, b1a, w1b, b1b)
    h = bconv(h, w2a, b2a, w2b, b2b)
    o_ref[...] = jnp.dot(h.astype(_BF), wl[...], preferred_element_type=_F32) + bl[...]


def _run_global_head(x, ws):
    return pl.pallas_call(
        functools.partial(_head_kernel, pool=True),
        out_shape=jax.ShapeDtypeStruct((8, 128), _F32),
        compiler_params=pltpu.CompilerParams(
            vmem_limit_bytes=48 * 1024 * 1024),
    )(x, *ws)


def _run_local_head(x, ws):
    B, H, W, C = x.shape
    Bb = 4
    M = Bb * H * W
    in_specs = [pl.BlockSpec((Bb, H, W, C), lambda i: (i, 0, 0, 0))]
    in_specs += [pl.BlockSpec(w.shape, lambda i: (0, 0)) for w in ws]
    return pl.pallas_call(
        functools.partial(_head_kernel, pool=False),
        out_shape=jax.ShapeDtypeStruct((B * H * W, 128), _F32),
        grid_spec=pltpu.PrefetchScalarGridSpec(
            num_scalar_prefetch=0, grid=(B // Bb,),
            in_specs=in_specs,
            out_specs=pl.BlockSpec((M, 128), lambda i: (i, 0))),
        compiler_params=pltpu.CompilerParams(
            dimension_semantics=("parallel",)),
    )(x, *ws)


_LAYER_CFG = ((0, 3, 1), (1, 4, 2), (2, 6, 2), (3, 3, 2))
_BB_FOR_LAYER = {0: 2, 1: 4, 2: 8, 3: 8}


def kernel(conv1_w, conv1_b, l0b0_conv1_w, l0b0_conv1_b, l0b0_conv2_w, l0b0_conv2_b, l0b0_conv3_w, l0b0_conv3_b, l0b0_down_w, l0b0_down_b, l0b1_conv1_w, l0b1_conv1_b, l0b1_conv2_w, l0b1_conv2_b, l0b1_conv3_w, l0b1_conv3_b, l0b2_conv1_w, l0b2_conv1_b, l0b2_conv2_w, l0b2_conv2_b, l0b2_conv3_w, l0b2_conv3_b, l1b0_conv1_w, l1b0_conv1_b, l1b0_conv2_w, l1b0_conv2_b, l1b0_conv3_w, l1b0_conv3_b, l1b0_down_w, l1b0_down_b, l1b1_conv1_w, l1b1_conv1_b, l1b1_conv2_w, l1b1_conv2_b, l1b1_conv3_w, l1b1_conv3_b, l1b2_conv1_w, l1b2_conv1_b, l1b2_conv2_w, l1b2_conv2_b, l1b2_conv3_w, l1b2_conv3_b, l1b3_conv1_w, l1b3_conv1_b, l1b3_conv2_w, l1b3_conv2_b, l1b3_conv3_w, l1b3_conv3_b, l2b0_conv1_w, l2b0_conv1_b, l2b0_conv2_w, l2b0_conv2_b, l2b0_conv3_w, l2b0_conv3_b, l2b0_down_w, l2b0_down_b, l2b1_conv1_w, l2b1_conv1_b, l2b1_conv2_w, l2b1_conv2_b, l2b1_conv3_w, l2b1_conv3_b, l2b2_conv1_w, l2b2_conv1_b, l2b2_conv2_w, l2b2_conv2_b, l2b2_conv3_w, l2b2_conv3_b, l2b3_conv1_w, l2b3_conv1_b, l2b3_conv2_w, l2b3_conv2_b, l2b3_conv3_w, l2b3_conv3_b, l2b4_conv1_w, l2b4_conv1_b, l2b4_conv2_w, l2b4_conv2_b, l2b4_conv3_w, l2b4_conv3_b, l2b5_conv1_w, l2b5_conv1_b, l2b5_conv2_w, l2b5_conv2_b, l2b5_conv3_w, l2b5_conv3_b, l3b0_conv1_w, l3b0_conv1_b, l3b0_conv2_w, l3b0_conv2_b, l3b0_conv3_w, l3b0_conv3_b, l3b0_down_w, l3b0_down_b, l3b1_conv1_w, l3b1_conv1_b, l3b1_conv2_w, l3b1_conv2_b, l3b1_conv3_w, l3b1_conv3_b, l3b2_conv1_w, l3b2_conv1_b, l3b2_conv2_w, l3b2_conv2_b, l3b2_conv3_w, l3b2_conv3_b, fc_bn1_c1_w, fc_bn1_c1_b, fc_bn1_c2_w, fc_bn1_c2_b, fc_bn2_c1_w, fc_bn2_c1_b, fc_bn2_c2_w, fc_bn2_c2_b, fc_lin_w, fc_lin_b, proj_bn1_c1_w, proj_bn1_c1_b, proj_bn1_c2_w, proj_bn1_c2_b, proj_bn2_c1_w, proj_bn2_c1_b, proj_bn2_c2_w, proj_bn2_c2_b, proj_conv_w, proj_conv_b, coord_obj, mask_obj):
    flat = dict(locals())

    x = (coord_obj * mask_obj).astype(_BF)
    z = x.reshape(8, 3, 64, 2, 64, 2).transpose(0, 2, 4, 3, 5, 1)
    z = z.reshape(8, 64, 64, 12)
    w7 = conv1_w[:147].reshape(7, 7, 3, 128)
    w7p = jnp.pad(w7, ((1, 0), (1, 0), (0, 0), (0, 0)))
    wz = w7p.reshape(4, 2, 4, 2, 3, 128).transpose(0, 2, 1, 3, 4, 5)
    wz = jnp.pad(wz.reshape(4, 4, 12, 128), ((0, 0), (0, 0), (0, 4), (0, 0)))
    wz = wz.reshape(256, 128)
    x = _run_stem(z, wz, conv1_b)

    seen = None
    for li, nb, s0 in _LAYER_CFG:
        blocks = []
        for bi in range(nb):
            pre = "l%db%d_" % (li, bi)
            blk = (flat[pre + "conv1_w"], flat[pre + "conv1_b"],
                   flat[pre + "conv2_w"], flat[pre + "conv2_b"],
                   flat[pre + "conv3_w"], flat[pre + "conv3_b"])
            if bi == 0:
                blk = blk + (flat[pre + "down_w"], flat[pre + "down_b"])
            blocks.append(blk)
        x = _run_layer(x, blocks, s0, _BB_FOR_LAYER[li])
        if li == 2:
            seen = x

    g = _run_global_head(
        x, (fc_bn1_c1_w, fc_bn1_c1_b, fc_bn1_c2_w, fc_bn1_c2_b,
            fc_bn2_c1_w, fc_bn2_c1_b, fc_bn2_c2_w, fc_bn2_c2_b,
            fc_lin_w, fc_lin_b))
    f = _run_local_head(
        seen, (proj_bn1_c1_w, proj_bn1_c1_b, proj_bn1_c2_w, proj_bn1_c2_b,
               proj_bn2_c1_w, proj_bn2_c1_b, proj_bn2_c2_w, proj_bn2_c2_b,
               proj_conv_w, proj_conv_b))
    return jnp.concatenate([g[:, None, :], f.reshape(8, 64, 128)], axis=1)


# speedup vs baseline: 1.0654x; 1.0024x over previous
import functools
import jax
import jax.numpy as jnp
from jax.experimental import pallas as pl
from jax.experimental.pallas import tpu as pltpu

_BF = jnp.bfloat16
_F32 = jnp.float32


def _dec2(v):
    b, h2, w2, c = v.shape
    v6 = v.reshape(b, h2 // 2, 2, w2 // 2, 2, c)
    return v6[:, :, 0, :, 0, :]


def _bneck(x, refs, s3_ref, H, W, stride, P, has_down):
    w1_ref, b1_ref, w2_ref, b2_ref, w3_ref, b3_ref = refs[:6]
    Bb, _, _, Cin = x.shape
    oh, ow = H // stride, W // stride
    xm = x.reshape(Bb * H * W, Cin)
    y1 = jnp.dot(xm, w1_ref[...], preferred_element_type=_F32) + b1_ref[...]
    y1 = jnp.maximum(y1, 0.0).astype(_BF).reshape(Bb, H, W, P)
    s3_ref[:, 0:H + 2, 0:W, :] = jnp.zeros((Bb, H + 2, W, 3 * P), _BF)
    s3_ref[:, 1:H + 1, 1:W, 0:P] = y1[:, :, 0:W - 1, :]
    s3_ref[:, 1:H + 1, 0:W, P:2 * P] = y1
    s3_ref[:, 1:H + 1, 0:W - 1, 2 * P:3 * P] = y1[:, :, 1:W, :]
    acc = b2_ref[...]
    for di in range(3):
        tap = s3_ref[:, di:di + H, 0:W, :].reshape(Bb * H * W, 3 * P)
        acc = acc + jnp.dot(tap, w2_ref[di * 3 * P:(di + 1) * 3 * P, :],
                            preferred_element_type=_F32)
    y2 = jnp.maximum(acc, 0.0).astype(_BF)
    if stride == 2:
        y2 = _dec2(y2.reshape(Bb, H, W, P)).reshape(Bb * oh * ow, P)
    z = jnp.dot(y2, w3_ref[...], preferred_element_type=_F32) + b3_ref[...]
    if has_down:
        wd_ref, bd_ref = refs[6:8]
        xs = _dec2(x) if stride == 2 else x
        scm = xs.reshape(Bb * oh * ow, Cin)
        z = z + jnp.dot(scm, wd_ref[...], preferred_element_type=_F32) + bd_ref[...]
    else:
        z = z + xm.astype(_F32)
    Cout = w3_ref.shape[1]
    return jnp.maximum(z, 0.0).astype(_BF).reshape(Bb, oh, ow, Cout)


def _layer_kernel(x_ref, *rest, H, W, s0, P, nb):
    refs = rest[:8 + 6 * (nb - 1)]
    o_ref, s3_ref = rest[8 + 6 * (nb - 1):]
    x = x_ref[...]
    x = _bneck(x, refs[0:8], s3_ref, H, W, s0, P, True)
    oh, ow = H // s0, W // s0
    for k in range(nb - 1):
        x = _bneck(x, refs[8 + 6 * k:8 + 6 * (k + 1)], s3_ref,
                   oh, ow, 1, P, False)
    o_ref[...] = x


def _run_layer(x, blocks, s0, Bb):
    B, H, W, Cin = x.shape
    P = blocks[0][0].shape[1]
    Cout = blocks[0][4].shape[1]
    oh, ow = H // s0, W // s0
    nb = len(blocks)
    args = [x] + [a for blk in blocks for a in blk]
    in_specs = [pl.BlockSpec((Bb, H, W, Cin), lambda i: (i, 0, 0, 0))]
    for blk in blocks:
        in_specs += [pl.BlockSpec(a.shape, lambda i: (0, 0)) for a in blk]
    kern = functools.partial(_layer_kernel, H=H, W=W, s0=s0, P=P, nb=nb)
    return pl.pallas_call(
        kern,
        out_shape=jax.ShapeDtypeStruct((B, oh, ow, Cout), _BF),
        grid_spec=pltpu.PrefetchScalarGridSpec(
            num_scalar_prefetch=0, grid=(B // Bb,),
            in_specs=in_specs,
            out_specs=pl.BlockSpec((Bb, oh, ow, Cout), lambda i: (i, 0, 0, 0)),
            scratch_shapes=[pltpu.VMEM((Bb, H + 2, W, 3 * P), _BF)]),
        compiler_params=pltpu.CompilerParams(
            dimension_semantics=("parallel",),
            vmem_limit_bytes=64 * 1024 * 1024),
    )(*args)


def _stem_kernel(z_ref, w_ref, b_ref, o_ref, s_ref, ip_ref, jp_ref):
    zz = z_ref[...]
    Bb = zz.shape[0]
    s_ref[...] = jnp.zeros((Bb, 67, 64, 64), _BF)
    s_ref[:, 2:66, 2:64, 0:12] = zz[:, :, 0:62, :]
    s_ref[:, 2:66, 1:64, 16:28] = zz[:, :, 0:63, :]
    s_ref[:, 2:66, 0:64, 32:44] = zz
    s_ref[:, 2:66, 0:63, 48:60] = zz[:, :, 1:64, :]
    acc = b_ref[...]
    for q in range(4):
        tap = s_ref[:, q:q + 64, :, :].reshape(Bb * 64 * 64, 64)
        acc = acc + jnp.dot(tap, w_ref[q * 64:(q + 1) * 64, :],
                            preferred_element_type=_F32)
    y = jnp.maximum(acc, 0.0).astype(_BF).reshape(Bb, 64, 64, 128)
    ip_ref[...] = jnp.full((Bb, 66, 64, 128), -jnp.inf, _BF)
    ip_ref[:, 1:65, :, :] = y
    a = jnp.maximum(jnp.maximum(ip_ref[:, 0:64, :, :], ip_ref[:, 1:65, :, :]),
                    ip_ref[:, 2:66, :, :])
    d = a.reshape(Bb, 32, 2, 64, 128)[:, :, 0]
    jp_ref[...] = jnp.full((Bb, 32, 66, 128), -jnp.inf, _BF)
    jp_ref[:, :, 1:65, :] = d
    e = jp_ref[...].reshape(Bb, 32, 33, 2, 128)
    out = jnp.maximum(jnp.maximum(e[:, :, 0:32, 0], e[:, :, 0:32, 1]),
                      e[:, :, 1:33, 0])
    o_ref[...] = out.reshape(Bb, 32, 32, 128)


def _run_stem(z, wz, b, Bb=1):
    B = z.shape[0]
    return pl.pallas_call(
        _stem_kernel,
        out_shape=jax.ShapeDtypeStruct((B, 32, 32, 128), _BF),
        grid_spec=pltpu.PrefetchScalarGridSpec(
            num_scalar_prefetch=0, grid=(B // Bb,),
            in_specs=[pl.BlockSpec((Bb, 64, 64, 12), lambda i: (i, 0, 0, 0)),
                      pl.BlockSpec(wz.shape, lambda i: (0, 0)),
                      pl.BlockSpec(b.shape, lambda i: (0, 0))],
            out_specs=pl.BlockSpec((Bb, 32, 32, 128), lambda i: (i, 0, 0, 0)),
            scratch_shapes=[pltpu.VMEM((Bb, 67, 64, 64), _BF),
                            pltpu.VMEM((Bb, 66, 64, 128), _BF),
                            pltpu.VMEM((Bb, 32, 66, 128), _BF)]),
        compiler_params=pltpu.CompilerParams(
            dimension_semantics=("parallel",)),
    )(z, wz, b)


def _head_kernel(x_ref, w1a, b1a, w1b, b1b, w2a, b2a, w2b, b2b, wl, bl,
                 o_ref, *, pool):
    if pool:
        xm = x_ref[...].reshape(8 * 16, 2048).astype(_F32)
        h = jnp.mean(xm.reshape(8, 16, 2048), axis=1)
    else:
        s = x_ref.shape
        h = x_ref[...].reshape(s[0] * s[1] * s[2], s[3]).astype(_F32)

    def bconv(hh, wa, ba, wb, bb):
        y = jnp.dot(hh.astype(_BF), wa[...], preferred_element_type=_F32) + ba[...]
        y = jnp.maximum(y, 0.0)
        zz = jnp.dot(y.astype(_BF), wb[...], preferred_element_type=_F32) + bb[...]
        return jnp.maximum(zz + hh, 0.0)

    h = bconv(h, w1a, b1a, w1b, b1b)
    h = bconv(h, w2a, b2a, w2b, b2b)
    o_ref[...] = jnp.dot(h.astype(_BF), wl[...], preferred_element_type=_F32) + bl[...]


def _run_global_head(x, ws):
    return pl.pallas_call(
        functools.partial(_head_kernel, pool=True),
        out_shape=jax.ShapeDtypeStruct((8, 128), _F32),
        compiler_params=pltpu.CompilerParams(
            vmem_limit_bytes=48 * 1024 * 1024),
    )(x, *ws)


def _run_local_head(x, ws):
    B, H, W, C = x.shape
    Bb = 4
    M = Bb * H * W
    in_specs = [pl.BlockSpec((Bb, H, W, C), lambda i: (i, 0, 0, 0))]
    in_specs += [pl.BlockSpec(w.shape, lambda i: (0, 0)) for w in ws]
    return pl.pallas_call(
        functools.partial(_head_kernel, pool=False),
        out_shape=jax.ShapeDtypeStruct((B * H * W, 128), _F32),
        grid_spec=pltpu.PrefetchScalarGridSpec(
            num_scalar_prefetch=0, grid=(B // Bb,),
            in_specs=in_specs,
            out_specs=pl.BlockSpec((M, 128), lambda i: (i, 0))),
        compiler_params=pltpu.CompilerParams(
            dimension_semantics=("parallel",)),
    )(x, *ws)


_LAYER_CFG = ((0, 3, 1), (1, 4, 2), (2, 6, 2), (3, 3, 2))
_BB_FOR_LAYER = {0: 4, 1: 4, 2: 8, 3: 8}


def kernel(conv1_w, conv1_b, l0b0_conv1_w, l0b0_conv1_b, l0b0_conv2_w, l0b0_conv2_b, l0b0_conv3_w, l0b0_conv3_b, l0b0_down_w, l0b0_down_b, l0b1_conv1_w, l0b1_conv1_b, l0b1_conv2_w, l0b1_conv2_b, l0b1_conv3_w, l0b1_conv3_b, l0b2_conv1_w, l0b2_conv1_b, l0b2_conv2_w, l0b2_conv2_b, l0b2_conv3_w, l0b2_conv3_b, l1b0_conv1_w, l1b0_conv1_b, l1b0_conv2_w, l1b0_conv2_b, l1b0_conv3_w, l1b0_conv3_b, l1b0_down_w, l1b0_down_b, l1b1_conv1_w, l1b1_conv1_b, l1b1_conv2_w, l1b1_conv2_b, l1b1_conv3_w, l1b1_conv3_b, l1b2_conv1_w, l1b2_conv1_b, l1b2_conv2_w, l1b2_conv2_b, l1b2_conv3_w, l1b2_conv3_b, l1b3_conv1_w, l1b3_conv1_b, l1b3_conv2_w, l1b3_conv2_b, l1b3_conv3_w, l1b3_conv3_b, l2b0_conv1_w, l2b0_conv1_b, l2b0_conv2_w, l2b0_conv2_b, l2b0_conv3_w, l2b0_conv3_b, l2b0_down_w, l2b0_down_b, l2b1_conv1_w, l2b1_conv1_b, l2b1_conv2_w, l2b1_conv2_b, l2b1_conv3_w, l2b1_conv3_b, l2b2_conv1_w, l2b2_conv1_b, l2b2_conv2_w, l2b2_conv2_b, l2b2_conv3_w, l2b2_conv3_b, l2b3_conv1_w, l2b3_conv1_b, l2b3_conv2_w, l2b3_conv2_b, l2b3_conv3_w, l2b3_conv3_b, l2b4_conv1_w, l2b4_conv1_b, l2b4_conv2_w, l2b4_conv2_b, l2b4_conv3_w, l2b4_conv3_b, l2b5_conv1_w, l2b5_conv1_b, l2b5_conv2_w, l2b5_conv2_b, l2b5_conv3_w, l2b5_conv3_b, l3b0_conv1_w, l3b0_conv1_b, l3b0_conv2_w, l3b0_conv2_b, l3b0_conv3_w, l3b0_conv3_b, l3b0_down_w, l3b0_down_b, l3b1_conv1_w, l3b1_conv1_b, l3b1_conv2_w, l3b1_conv2_b, l3b1_conv3_w, l3b1_conv3_b, l3b2_conv1_w, l3b2_conv1_b, l3b2_conv2_w, l3b2_conv2_b, l3b2_conv3_w, l3b2_conv3_b, fc_bn1_c1_w, fc_bn1_c1_b, fc_bn1_c2_w, fc_bn1_c2_b, fc_bn2_c1_w, fc_bn2_c1_b, fc_bn2_c2_w, fc_bn2_c2_b, fc_lin_w, fc_lin_b, proj_bn1_c1_w, proj_bn1_c1_b, proj_bn1_c2_w, proj_bn1_c2_b, proj_bn2_c1_w, proj_bn2_c1_b, proj_bn2_c2_w, proj_bn2_c2_b, proj_conv_w, proj_conv_b, coord_obj, mask_obj):
    flat = dict(locals())

    x = (coord_obj * mask_obj).astype(_BF)
    z = x.reshape(8, 3, 64, 2, 64, 2).transpose(0, 2, 4, 3, 5, 1)
    z = z.reshape(8, 64, 64, 12)
    w7 = conv1_w[:147].reshape(7, 7, 3, 128)
    w7p = jnp.pad(w7, ((1, 0), (1, 0), (0, 0), (0, 0)))
    wz = w7p.reshape(4, 2, 4, 2, 3, 128).transpose(0, 2, 1, 3, 4, 5)
    wz = jnp.pad(wz.reshape(4, 4, 12, 128), ((0, 0), (0, 0), (0, 4), (0, 0)))
    wz = wz.reshape(256, 128)
    x = _run_stem(z, wz, conv1_b)

    seen = None
    for li, nb, s0 in _LAYER_CFG:
        blocks = []
        for bi in range(nb):
            pre = "l%db%d_" % (li, bi)
            blk = (flat[pre + "conv1_w"], flat[pre + "conv1_b"],
                   flat[pre + "conv2_w"], flat[pre + "conv2_b"],
                   flat[pre + "conv3_w"], flat[pre + "conv3_b"])
            if bi == 0:
                blk = blk + (flat[pre + "down_w"], flat[pre + "down_b"])
            blocks.append(blk)
        x = _run_layer(x, blocks, s0, _BB_FOR_LAYER[li])
        if li == 2:
            seen = x

    g = _run_global_head(
        x, (fc_bn1_c1_w, fc_bn1_c1_b, fc_bn1_c2_w, fc_bn1_c2_b,
            fc_bn2_c1_w, fc_bn2_c1_b, fc_bn2_c2_w, fc_bn2_c2_b,
            fc_lin_w, fc_lin_b))
    f = _run_local_head(
        seen, (proj_bn1_c1_w, proj_bn1_c1_b, proj_bn1_c2_w, proj_bn1_c2_b,
               proj_bn2_c1_w, proj_bn2_c1_b, proj_bn2_c2_w, proj_bn2_c2_b,
               proj_conv_w, proj_conv_b))
    return jnp.concatenate([g[:, None, :], f.reshape(8, 64, 128)], axis=1)
